# Initial kernel scaffold; baseline (speedup 1.0000x reference)
import jax
import jax.numpy as jnp
from jax.experimental import pallas as pl


def kernel(x, edge_index, batch, W1, W2):
    raise NotImplementedError("write your pallas kernel here")


# speedup vs baseline: 4.3054x; 4.3054x over previous
import functools

import jax
import jax.numpy as jnp
from jax import lax
from jax.experimental import pallas as pl
from jax.experimental.pallas import tpu as pltpu
from jax.experimental.pallas import tpu_sc as plsc

_N = 10000
_E = 320000
_G = 64
_F = 128
_C = 40
_CP = 128

_NC, _NS = 2, 16
_NW = _NC * _NS
_CH = 128
_NCHUNK = 79
_EPW = _NCHUNK * _CH
_EPAD = _NW * _EPW
_NPAD = 10240
_ZCH = 128
_KPT = _NPAD // _NS // _ZCH


def _make_edge_agg(d):
  mesh = plsc.VectorSubcoreMesh(core_axis_name="c", subcore_axis_name="s")

  @functools.partial(
      pl.kernel,
      mesh=mesh,
      out_type=jax.ShapeDtypeStruct((_NC, _NPAD, d), jnp.float32),
      scratch_types=[
          pltpu.VMEM((_CH,), jnp.int32),
          pltpu.VMEM((_CH,), jnp.int32),
          pltpu.VMEM((_CH, d), jnp.float32),
          pltpu.VMEM((_ZCH, d), jnp.float32),
          pltpu.VMEM_SHARED((_NPAD, d), jnp.float32),
          pltpu.SemaphoreType.DMA,
      ],
  )
  def agg(table_hbm, src_hbm, dst_hbm, out_hbm,
          src_v, dst_v, msgs_v, zbuf_v, acc_sh, sem):
    c = lax.axis_index("c")
    s = lax.axis_index("s")
    wid = s * _NC + c

    def _zrow(i, carry):
      for j in range(d // 16):
        zbuf_v[i, pl.ds(j * 16, 16)] = jnp.zeros((16,), jnp.float32)
      return carry

    lax.fori_loop(0, _ZCH, _zrow, 0)
    for k in range(_KPT):
      pltpu.sync_copy(zbuf_v, acc_sh.at[pl.ds((s * _KPT + k) * _ZCH, _ZCH)])
    plsc.subcore_barrier()

    ebase = wid * _EPW

    def _edges(j, carry):
      b = ebase + j * _CH
      pltpu.sync_copy(src_hbm.at[pl.ds(b, _CH)], src_v)
      pltpu.sync_copy(dst_hbm.at[pl.ds(b, _CH)], dst_v)
      pltpu.async_copy(table_hbm.at[src_v], msgs_v, sem).wait()
      pltpu.sync_copy(msgs_v, acc_sh.at[dst_v], add=True)
      return carry

    lax.fori_loop(0, _NCHUNK, _edges, 0)
    plsc.subcore_barrier()

    for k in range(_KPT):
      r0 = (s * _KPT + k) * _ZCH
      pltpu.sync_copy(acc_sh.at[pl.ds(r0, _ZCH)], zbuf_v)
      pltpu.sync_copy(zbuf_v, out_hbm.at[c, pl.ds(r0, _ZCH)])

  return agg


_agg_x = _make_edge_agg(_F)
_agg_q = _agg_x if _CP == _F else _make_edge_agg(_CP)


def _tc_transform(p, w1, w2p):
  def body(p0, p1, a, b, o):
    t = jnp.dot(p0[...] + p1[...], a[...], preferred_element_type=jnp.float32)
    t = jnp.maximum(t, 0.0)
    o[...] = jnp.dot(t, b[...], preferred_element_type=jnp.float32)

  blk = 1280
  return pl.pallas_call(
      body,
      grid=(_NPAD // blk,),
      in_specs=[
          pl.BlockSpec((None, blk, _F), lambda i: (0, i, 0)),
          pl.BlockSpec((None, blk, _F), lambda i: (1, i, 0)),
          pl.BlockSpec((_F, _F), lambda i: (0, 0)),
          pl.BlockSpec((_F, _CP), lambda i: (0, 0)),
      ],
      out_specs=pl.BlockSpec((blk, _CP), lambda i: (i, 0)),
      out_shape=jax.ShapeDtypeStruct((_NPAD, _CP), jnp.float32),
  )(p, p, w1, w2p)


def _tc_pool(p2, batch2d):
  def body(p0, p1, b, o):
    h2 = p0[...] + p1[...]
    gids = lax.broadcasted_iota(jnp.int32, (_NPAD, _G), 1)
    onehot = jnp.where(b[...] == gids, 1.0, 0.0).astype(jnp.float32)
    sums = lax.dot_general(onehot, h2, (((0,), (0,)), ((), ())),
                           preferred_element_type=jnp.float32)
    counts = jnp.maximum(jnp.sum(onehot, axis=0), 1.0)
    o[...] = sums / counts[:, None]

  return pl.pallas_call(
      body,
      grid=(1,),
      in_specs=[
          pl.BlockSpec((None, _NPAD, _CP), lambda i: (0, 0, 0)),
          pl.BlockSpec((None, _NPAD, _CP), lambda i: (1, 0, 0)),
          pl.BlockSpec((_NPAD, 1), lambda i: (0, 0)),
      ],
      out_specs=pl.BlockSpec((_G, _CP), lambda i: (0, 0)),
      out_shape=jax.ShapeDtypeStruct((_G, _CP), jnp.float32),
  )(p2, p2, batch2d)


def kernel(x, edge_index, batch, W1, W2):
  src = edge_index[0].astype(jnp.int32)
  dst = edge_index[1].astype(jnp.int32)
  pad = _EPAD - _E
  src_p = jnp.concatenate([src, jnp.zeros((pad,), jnp.int32)])
  dst_p = jnp.concatenate([dst, jnp.full((pad,), _N, jnp.int32)])

  p1 = _agg_x(x, src_p, dst_p)
  w2p = jnp.pad(W2, ((0, 0), (0, _CP - _C)))
  q = _tc_transform(p1, W1, w2p)
  p2 = _agg_q(q, src_p, dst_p)

  bpad = jnp.concatenate(
      [batch.astype(jnp.int32), jnp.full((_NPAD - _N,), _G, jnp.int32)])
  out = _tc_pool(p2, bpad.reshape(_NPAD, 1))
  return out[:, :_C]
